# baseline (speedup 1.0000x reference)
import functools

import jax
import jax.numpy as jnp
from jax import lax
from jax.experimental import pallas as pl
from jax.experimental.pallas import tpu as pltpu
from jax.experimental.pallas import tpu_sc as plsc

_B, _N, _M = 4, 4096, 4096
_NTC = 3456
_TN = 1728
_NT = _NTC // _TN
_NSC = _N - _NTC
_RPW = (_B * _NSC) // 32
_WPB = 8
_CPW = _M // _WPB
_C1 = 1000.0 / (2.0 * _B * _N)
_C2 = 1000.0 / (2.0 * _B * _M)
_L = 16
_BIG = 3.0e38


def _tc_body(a1_ref, a2t_ref, s1_ref, d2_ref, d2_scr):
    n = pl.program_id(1)

    a1 = a1_ref[0]
    a1x = a1[:, 0:1]
    a1y = a1[:, 1:2]
    a1z = a1[:, 2:3]
    asq = a1x * a1x + a1y * a1y + a1z * a1z

    a2t = a2t_ref[0]
    a2x = a2t[0:1, :]
    a2y = a2t[1:2, :]
    a2z = a2t[2:3, :]
    bsq = a2x * a2x + a2y * a2y + a2z * a2z

    u = jax.lax.dot_general(
        a1.astype(jnp.bfloat16),
        a2t.astype(jnp.bfloat16) * jnp.bfloat16(-2.0),
        (((1,), (0,)), ((), ())),
        preferred_element_type=jnp.float32,
    )
    d = (asq + bsq) + u

    @pl.when(jnp.logical_and(pl.program_id(0) == 0, n == 0))
    def _():
        s1_ref[...] = jnp.zeros((1, 1), jnp.float32)

    d1 = jnp.maximum(jnp.min(d, axis=1, keepdims=True), 0.0)
    s1_ref[...] += jnp.sum(jnp.sqrt(d1), keepdims=True)

    dmin = jnp.min(d, axis=0, keepdims=True)

    @pl.when(n == 0)
    def _():
        d2_scr[...] = dmin

    @pl.when(n > 0)
    def _():
        d2_scr[...] = jnp.minimum(d2_scr[...], dmin)

    @pl.when(n == _NT - 1)
    def _():
        d2_ref[...] = d2_scr[...][None]


def _tc_part(a1_tc, a2t):
    return pl.pallas_call(
        _tc_body,
        grid=(_B, _NT),
        in_specs=[
            pl.BlockSpec((1, _TN, 3), lambda b, n: (b, n, 0)),
            pl.BlockSpec((1, 3, _M), lambda b, n: (b, 0, 0)),
        ],
        out_specs=[
            pl.BlockSpec((1, 1), lambda b, n: (0, 0)),
            pl.BlockSpec((1, 1, _M), lambda b, n: (b, 0, 0)),
        ],
        out_shape=[
            jax.ShapeDtypeStruct((1, 1), jnp.float32),
            jax.ShapeDtypeStruct((_B, 1, _M), jnp.float32),
        ],
        scratch_shapes=[pltpu.VMEM((1, _M), jnp.float32)],
    )(a1_tc, a2t)


def _vsqrt(x):
    x = jnp.maximum(x, 0.0)
    i = plsc.bitcast(x, jnp.int32)
    i = jnp.int32(0x5F3759DF) - jnp.right_shift(i, jnp.int32(1))
    y = plsc.bitcast(i, jnp.float32)
    half_x = 0.5 * x
    for _ in range(3):
        y = y * (1.5 - half_x * y * y)
    return x * y


def _sc_body(a1x, a1y, a1z, a2x, a2y, a2z, out1, out2,
             r1x, r1y, r1z, sq1,
             f2x, f2y, f2z, r2x, r2y, r2z, bsq,
             cmin, rmin, comb, ctmp, accv, shared):
    c = lax.axis_index("c")
    s = lax.axis_index("s")
    batch = c * 2 + s // _WPB
    row0 = batch * _N + _NTC + (s % _WPB) * _RPW
    pltpu.sync_copy(a1x.at[pl.ds(row0, _RPW)], r1x)
    pltpu.sync_copy(a1y.at[pl.ds(row0, _RPW)], r1y)
    pltpu.sync_copy(a1z.at[pl.ds(row0, _RPW)], r1z)
    col0 = batch * _M
    pltpu.sync_copy(a2x.at[pl.ds(col0, _M)], f2x)
    pltpu.sync_copy(a2y.at[pl.ds(col0, _M)], f2y)
    pltpu.sync_copy(a2z.at[pl.ds(col0, _M)], f2z)

    def _rb(v):
        i = plsc.bitcast(v, jnp.uint32)
        lsb = jnp.bitwise_and(jnp.right_shift(i, jnp.uint32(16)), jnp.uint32(1))
        r = jnp.bitwise_and(i + jnp.uint32(0x7FFF) + lsb, jnp.uint32(0xFFFF0000))
        return plsc.bitcast(r, jnp.float32)

    def _prep1(k, carry):
        sl = pl.ds(k * _L, _L)
        x = r1x[sl]
        y = r1y[sl]
        z = r1z[sl]
        sq1[sl] = x * x + y * y + z * z
        r1x[sl] = _rb(x)
        r1y[sl] = _rb(y)
        r1z[sl] = _rb(z)
        return carry

    lax.fori_loop(0, _RPW // _L, _prep1, 0, unroll=2)

    def _prep2(k, carry):
        sl = pl.ds(k * _L, _L)
        x = f2x[sl]
        y = f2y[sl]
        z = f2z[sl]
        bsq[sl] = x * x + y * y + z * z
        r2x[sl] = jnp.float32(-2.0) * _rb(x)
        r2y[sl] = jnp.float32(-2.0) * _rb(y)
        r2z[sl] = jnp.float32(-2.0) * _rb(z)
        cmin[sl] = jnp.full((_L,), _BIG, jnp.float32)
        return carry

    lax.fori_loop(0, _M // _L, _prep2, 0, unroll=4)

    def _rowblock(rb, carry):
        base = rb * 8
        splats = []
        for r in range(8):
            idx = jnp.full((_L,), base + r, jnp.int32)
            splats.append((
                plsc.load_gather(r1x, [idx]),
                plsc.load_gather(r1y, [idx]),
                plsc.load_gather(r1z, [idx]),
                plsc.load_gather(sq1, [idx]),
            ))

        def _colblock(jb, rms):
            sl = pl.ds(jb * _L, _L)
            bx = r2x[sl]
            by = r2y[sl]
            bz = r2z[sl]
            bq = bsq[sl]
            cm = cmin[sl]
            new = []
            for r in range(8):
                ax, ay, az, aq = splats[r]
                cr = ax * bx + ay * by + az * bz
                d = (aq + bq) + cr
                new.append(jnp.minimum(rms[r], d))
                cm = jnp.minimum(cm, d)
            cmin[sl] = cm
            return tuple(new)

        rms = lax.fori_loop(
            0, _M // _L, _colblock,
            tuple(jnp.full((_L,), _BIG, jnp.float32) for _ in range(8)),
        )
        lane0 = lax.iota(jnp.int32, _L) == 0
        for r in range(8):
            m = jnp.min(rms[r])
            idx = jnp.full((_L,), base + r, jnp.int32)
            plsc.store_scatter(rmin, [idx], jnp.full((_L,), m), mask=lane0)
        return carry

    lax.fori_loop(0, _RPW // 8, _rowblock, 0)

    pltpu.sync_copy(cmin, shared.at[s])
    plsc.subcore_barrier()
    cbase = (s % _WPB) * _CPW
    peer0 = (s // _WPB) * _WPB

    def _init_comb(k, carry):
        comb[pl.ds(k * _L, _L)] = jnp.full((_L,), _BIG, jnp.float32)
        return carry

    lax.fori_loop(0, _CPW // _L, _init_comb, 0, unroll=4)

    def _peer(q, carry):
        pltpu.sync_copy(shared.at[peer0 + q, pl.ds(cbase, _CPW)], ctmp)

        def _mn(k, c2):
            sl = pl.ds(k * _L, _L)
            comb[sl] = jnp.minimum(comb[sl], ctmp[sl])
            return c2

        lax.fori_loop(0, _CPW // _L, _mn, 0, unroll=4)
        return carry

    lax.fori_loop(0, _WPB, _peer, 0)

    def _sum1(k, acc):
        return acc + _vsqrt(rmin[pl.ds(k * _L, _L)])

    acc1 = lax.fori_loop(0, _RPW // _L, _sum1,
                         jnp.zeros((_L,), jnp.float32), unroll=2)
    accv[0, :] = acc1
    wid = c * 16 + s
    pltpu.sync_copy(accv, out1.at[pl.ds(wid, 1)])
    pltpu.sync_copy(comb, out2.at[batch, 0, pl.ds(cbase, _CPW)])


def _sc_part(a1p, a2p):
    mesh = plsc.VectorSubcoreMesh(core_axis_name="c", subcore_axis_name="s")
    f = functools.partial(
        pl.kernel,
        mesh=mesh,
        compiler_params=pltpu.CompilerParams(needs_layout_passes=False),
        out_type=[
            jax.ShapeDtypeStruct((32, _L), jnp.float32),
            jax.ShapeDtypeStruct((_B, 1, _M), jnp.float32),
        ],
        scratch_types=[
            pltpu.VMEM((_RPW,), jnp.float32),
            pltpu.VMEM((_RPW,), jnp.float32),
            pltpu.VMEM((_RPW,), jnp.float32),
            pltpu.VMEM((_RPW,), jnp.float32),
            pltpu.VMEM((_M,), jnp.float32),
            pltpu.VMEM((_M,), jnp.float32),
            pltpu.VMEM((_M,), jnp.float32),
            pltpu.VMEM((_M,), jnp.float32),
            pltpu.VMEM((_M,), jnp.float32),
            pltpu.VMEM((_M,), jnp.float32),
            pltpu.VMEM((_M,), jnp.float32),
            pltpu.VMEM((_M,), jnp.float32),
            pltpu.VMEM((_RPW,), jnp.float32),
            pltpu.VMEM((_CPW,), jnp.float32),
            pltpu.VMEM((_CPW,), jnp.float32),
            pltpu.VMEM((1, _L), jnp.float32),
            pltpu.VMEM_SHARED((16, _M), jnp.float32),
        ],
    )(_sc_body)
    return f(a1p[0], a1p[1], a1p[2], a2p[0], a2p[1], a2p[2])


def _comb_body(s1tc_ref, s1sc_ref, d2a_ref, d2b_ref, out_ref):
    d2 = jnp.minimum(d2a_ref[...], d2b_ref[...])
    d2 = jnp.maximum(d2, 0.0)
    s2 = jnp.sum(jnp.sqrt(d2), keepdims=False)
    s1 = s1tc_ref[0, 0] + jnp.sum(s1sc_ref[...])
    out_ref[...] = jnp.full((1, 1), s1 * _C1, jnp.float32) + s2 * _C2


def _combine(s1tc, s1sc, d2a, d2b):
    return pl.pallas_call(
        _comb_body,
        out_shape=jax.ShapeDtypeStruct((1, 1), jnp.float32),
    )(s1tc, s1sc, d2a, d2b)


def kernel(array1, array2):
    a2t = jnp.transpose(array2, (0, 2, 1))
    a1_tc = array1[:, :_NTC]
    a1p = jnp.transpose(array1, (2, 0, 1)).reshape(3, _B * _N)
    a2p = jnp.transpose(array2, (2, 0, 1)).reshape(3, _B * _M)

    s1sc, d2sc = _sc_part(a1p, a2p)
    s1tc, d2tc = _tc_part(a1_tc, a2t)
    out = _combine(s1tc, s1sc, d2tc, d2sc)
    return out[0, 0]


# speedup vs baseline: 1.1108x; 1.1108x over previous
import jax
import jax.numpy as jnp
from jax.experimental import pallas as pl
from jax.experimental.pallas import tpu as pltpu

_B, _N, _M = 4, 4096, 4096
_TN = 2048
_NT = _N // _TN
_C1 = 1000.0 / (2.0 * _B * _N)
_C2 = 1000.0 / (2.0 * _B * _M)


def _chamfer_body(a1_ref, a2t_ref, out_ref, d2_scr):
    b = pl.program_id(0)
    n = pl.program_id(1)

    a1 = a1_ref[0]
    a1x = a1[:, 0:1]
    a1y = a1[:, 1:2]
    a1z = a1[:, 2:3]
    asq = a1x * a1x + a1y * a1y + a1z * a1z

    a2t = a2t_ref[0]
    a2x = a2t[0:1, :]
    a2y = a2t[1:2, :]
    a2z = a2t[2:3, :]
    bsq = a2x * a2x + a2y * a2y + a2z * a2z

    u = jax.lax.dot_general(
        a1.astype(jnp.bfloat16),
        a2t.astype(jnp.bfloat16) * jnp.bfloat16(-2.0),
        (((1,), (0,)), ((), ())),
        preferred_element_type=jnp.float32,
    )
    w = bsq + u

    @pl.when(jnp.logical_and(b == 0, n == 0))
    def _():
        out_ref[...] = jnp.zeros((1, 1), jnp.float32)

    m1 = jnp.min(w, axis=1, keepdims=True)
    d1 = jnp.maximum(m1 + asq, 0.0)
    s1 = jnp.sum(jnp.sqrt(d1), keepdims=True)

    dmin = jnp.min(asq + w, axis=0, keepdims=True)

    @pl.when(n == 0)
    def _():
        d2_scr[...] = dmin

    @pl.when(n > 0)
    def _():
        d2_scr[...] = jnp.minimum(d2_scr[...], dmin)

    out_ref[...] += s1 * _C1

    @pl.when(n == _NT - 1)
    def _():
        d2 = jnp.maximum(d2_scr[...], 0.0)
        out_ref[...] += jnp.sum(jnp.sqrt(d2), keepdims=True) * _C2


def kernel(array1, array2):
    a2t = jnp.transpose(array2, (0, 2, 1))
    out = pl.pallas_call(
        _chamfer_body,
        grid=(_B, _NT),
        in_specs=[
            pl.BlockSpec((1, _TN, 3), lambda b, n: (b, n, 0)),
            pl.BlockSpec((1, 3, _M), lambda b, n: (b, 0, 0)),
        ],
        out_specs=pl.BlockSpec((1, 1), lambda b, n: (0, 0)),
        out_shape=jax.ShapeDtypeStruct((1, 1), jnp.float32),
        scratch_shapes=[pltpu.VMEM((1, _M), jnp.float32)],
    )(array1, a2t)
    return out[0, 0]


# speedup vs baseline: 1.1882x; 1.0697x over previous
import jax
import jax.numpy as jnp
from jax.experimental import pallas as pl
from jax.experimental.pallas import tpu as pltpu

_B, _N, _M = 4, 4096, 4096
_TN = 2048
_NT = _N // _TN
_C1 = 1000.0 / (2.0 * _B * _N)
_C2 = 1000.0 / (2.0 * _B * _M)


def _chamfer_body(a1_ref, a2t_ref, out_ref, d2_scr):
    b = pl.program_id(0)
    n = pl.program_id(1)

    a1 = a1_ref[0]
    a1x = a1[:, 0:1]
    a1y = a1[:, 1:2]
    a1z = a1[:, 2:3]
    asq = a1x * a1x + a1y * a1y + a1z * a1z

    a2t = a2t_ref[0]
    a2x = a2t[0:1, :]
    a2y = a2t[1:2, :]
    a2z = a2t[2:3, :]
    bsq = a2x * a2x + a2y * a2y + a2z * a2z

    u = jax.lax.dot_general(
        a1.astype(jnp.bfloat16),
        a2t.astype(jnp.bfloat16) * jnp.bfloat16(-2.0),
        (((1,), (0,)), ((), ())),
        preferred_element_type=jnp.float32,
    )
    d = (asq + bsq) + u

    @pl.when(jnp.logical_and(b == 0, n == 0))
    def _():
        out_ref[...] = jnp.zeros((1, 1), jnp.float32)

    d1 = jnp.maximum(jnp.min(d, axis=1, keepdims=True), 0.0)
    s1 = jnp.sum(jnp.sqrt(d1), keepdims=True)

    dmin = jnp.min(d, axis=0, keepdims=True)

    @pl.when(n == 0)
    def _():
        d2_scr[...] = dmin

    @pl.when(n > 0)
    def _():
        d2_scr[...] = jnp.minimum(d2_scr[...], dmin)

    out_ref[...] += s1 * _C1

    @pl.when(n == _NT - 1)
    def _():
        d2 = jnp.maximum(d2_scr[...], 0.0)
        out_ref[...] += jnp.sum(jnp.sqrt(d2), keepdims=True) * _C2


def kernel(array1, array2):
    a2t = jnp.transpose(array2, (0, 2, 1))
    out = pl.pallas_call(
        _chamfer_body,
        grid=(_B, _NT),
        in_specs=[
            pl.BlockSpec((1, _TN, 3), lambda b, n: (b, n, 0)),
            pl.BlockSpec((1, 3, _M), lambda b, n: (b, 0, 0)),
        ],
        out_specs=pl.BlockSpec((1, 1), lambda b, n: (0, 0)),
        out_shape=jax.ShapeDtypeStruct((1, 1), jnp.float32),
        scratch_shapes=[pltpu.VMEM((1, _M), jnp.float32)],
    )(array1, a2t)
    return out[0, 0]
